# Initial kernel scaffold; baseline (speedup 1.0000x reference)
import jax
import jax.numpy as jnp
from jax.experimental import pallas as pl


def kernel(state, x, edge_index, edge_weight, batch, emb_table, W1, b1, W2, b2, fcW, fcb, fc1W, fc1b, fc2W, fc2b, fc3W, fc3b):
    raise NotImplementedError("write your pallas kernel here")


# speedup vs baseline: 6.4577x; 6.4577x over previous
import functools

import jax
import jax.numpy as jnp
from jax import lax
from jax.experimental import pallas as pl
from jax.experimental.pallas import tpu as pltpu, tpu_sc as plsc

N = 50000
NPAD = 50176
HALF = 25088
ACCR = 25120
E = 800000
EPAD = 802816
VOCAB = 100000
EMB = 256
HID = 64
B = 64

_MESH = plsc.VectorSubcoreMesh(core_axis_name="c", subcore_axis_name="s")
_SC_PARAMS = pltpu.CompilerParams(needs_layout_passes=False,
                                  use_tc_tiling_on_sc=False)


def _t1_body(emb_ref, w1_ref, out_ref):
    out_ref[...] = jnp.dot(emb_ref[...], w1_ref[...],
                           preferred_element_type=jnp.float32)


def _t1_matmul(emb, W1):
    bkv = 400
    return pl.pallas_call(
        _t1_body,
        grid=(VOCAB // bkv,),
        in_specs=[
            pl.BlockSpec((bkv, EMB), lambda i: (i, 0)),
            pl.BlockSpec((EMB, HID), lambda i: (0, 0)),
        ],
        out_specs=pl.BlockSpec((bkv, HID), lambda i: (i, 0)),
        out_shape=jax.ShapeDtypeStruct((VOCAB, HID), jnp.float32),
    )(emb, W1)


def _sc_pre_body(t1_hbm, x_hbm, dst_hbm, w_hbm, emb_hbm, sidx_hbm,
                 hw_hbm, degp_hbm, srows_hbm,
                 deg_v, xidx_v, rows_v, dstb_v, wb_v, sall_v, srow_v, sem):
    wid = lax.axis_index("s") * 2 + lax.axis_index("c")

    def zero_body(i, _):
        deg_v[pl.ds(i * 16, 16)] = jnp.zeros((16,), jnp.float32)
        return 0
    lax.fori_loop(0, NPAD // 16, zero_body, 0)

    ebase = wid * (EPAD // 32)
    nchunk = (EPAD // 32) // 128

    def deg_chunk(g, _):
        base = ebase + g * 128
        pltpu.sync_copy(dst_hbm.at[pl.ds(base, 128)], dstb_v)
        pltpu.sync_copy(w_hbm.at[pl.ds(base, 128)], wb_v)

        def deg_step(j, _):
            idx = dstb_v[pl.ds(j * 16, 16)]
            val = wb_v[pl.ds(j * 16, 16)]
            plsc.addupdate_scatter(deg_v, [idx], val)
            return 0
        lax.fori_loop(0, 8, deg_step, 0)
        return 0
    lax.fori_loop(0, nchunk, deg_chunk, 0)
    pltpu.sync_copy(deg_v, degp_hbm.at[wid])

    rbase = wid * 1568
    for j in range(6):
        xidx_v[pl.ds(1568 + j * 16, 16)] = jnp.zeros((16,), jnp.int32)
    pltpu.sync_copy(x_hbm.at[pl.ds(rbase, 1568)], xidx_v.at[pl.ds(0, 1568)])
    for i in range(13):
        sz = 128 if i < 12 else 32
        pltpu.async_copy(t1_hbm.at[xidx_v.at[pl.ds(i * 128, 128)]],
                         rows_v, sem).wait()
        pltpu.sync_copy(rows_v.at[pl.ds(0, sz)],
                        hw_hbm.at[pl.ds(rbase + i * 128, sz)])

    pltpu.sync_copy(sidx_hbm, sall_v)

    @pl.when(wid < 16)
    def _():
        pltpu.async_copy(emb_hbm.at[sall_v.at[pl.ds(wid * 8, 8)]],
                         srow_v, sem).wait()
        pltpu.sync_copy(srow_v, srows_hbm.at[pl.ds(wid * 8, 8)])


def _sc_pre(T1, xpad, dstp, wp, emb, sidx):
    f = pl.kernel(
        _sc_pre_body,
        out_type=(
            jax.ShapeDtypeStruct((NPAD, HID), jnp.float32),
            jax.ShapeDtypeStruct((32, NPAD), jnp.float32),
            jax.ShapeDtypeStruct((128, EMB), jnp.float32),
        ),
        mesh=_MESH,
        scratch_types=[
            pltpu.VMEM((NPAD,), jnp.float32),
            pltpu.VMEM((1664,), jnp.int32),
            pltpu.VMEM((128, HID), jnp.float32),
            pltpu.VMEM((128,), jnp.int32),
            pltpu.VMEM((128,), jnp.float32),
            pltpu.VMEM((128,), jnp.int32),
            pltpu.VMEM((8, EMB), jnp.float32),
            pltpu.SemaphoreType.DMA,
        ],
        compiler_params=_SC_PARAMS,
    )
    return f(T1, xpad, dstp, wp, emb, sidx)


def _sc_conv_body(hp_hbm, src_hbm, dst_hbm, w_hbm, out_hbm,
                  acc_sh, rows_v, srcb_v, dstb_v, wb_v, ldst_v, z_v, sem):
    c = lax.axis_index("c")
    s = lax.axis_index("s")

    def zb(i, _):
        for q in range(4):
            z_v[i, pl.ds(q * 16, 16)] = jnp.zeros((16,), jnp.float32)
        return 0
    lax.fori_loop(0, 128, zb, 0)
    zrows = ACCR // 16
    for i in range(13):
        sz = 128 if i < 12 else 34
        pltpu.sync_copy(z_v.at[pl.ds(0, sz)],
                        acc_sh.at[pl.ds(s * zrows + i * 128, sz)])
    plsc.subcore_barrier()

    ebase = s * (EPAD // 16)
    nchunk = (EPAD // 16) // 128

    def chunk(g, _):
        base = ebase + g * 128
        pltpu.sync_copy(src_hbm.at[pl.ds(base, 128)], srcb_v)
        pltpu.sync_copy(dst_hbm.at[pl.ds(base, 128)], dstb_v)
        pltpu.sync_copy(w_hbm.at[pl.ds(base, 128)], wb_v)
        pltpu.async_copy(hp_hbm.at[srcb_v], rows_v, sem).wait()

        def scale(r, _):
            wv = plsc.load_gather(wb_v, [jnp.full((16,), r, jnp.int32)])
            for q in range(4):
                rows_v[r, pl.ds(q * 16, 16)] = rows_v[r, pl.ds(q * 16, 16)] * wv
            return 0
        lax.fori_loop(0, 128, scale, 0)

        def lidx(j, _):
            d = dstb_v[pl.ds(j * 16, 16)] - c * HALF
            inb = (d >= 0) & (d < HALF)
            ldst_v[0, pl.ds(j * 16, 16)] = jnp.where(inb, d, HALF)
            return 0
        lax.fori_loop(0, 8, lidx, 0)

        pltpu.sync_copy(rows_v, acc_sh.at[ldst_v.at[0]], add=True)
        return 0
    lax.fori_loop(0, nchunk, chunk, 0)
    plsc.subcore_barrier()

    pltpu.sync_copy(acc_sh.at[pl.ds(s * 1568, 1568)],
                    out_hbm.at[c, pl.ds(s * 1568, 1568)])


def _sc_conv(hp, srcp, dstp, wp):
    f = pl.kernel(
        _sc_conv_body,
        out_type=jax.ShapeDtypeStruct((2, HALF, HID), jnp.float32),
        mesh=_MESH,
        scratch_types=[
            pltpu.VMEM_SHARED((ACCR, HID), jnp.float32),
            pltpu.VMEM((128, HID), jnp.float32),
            pltpu.VMEM((128,), jnp.int32),
            pltpu.VMEM((128,), jnp.int32),
            pltpu.VMEM((128,), jnp.float32),
            pltpu.VMEM((1, 128), jnp.int32),
            pltpu.VMEM((128, HID), jnp.float32),
            pltpu.SemaphoreType.DMA,
        ],
        compiler_params=_SC_PARAMS,
    )
    return f(hp, srcp, dstp, wp)


def _prep_body(degp_ref, hw_ref, dinvb_ref, hwp_ref):
    ones = jnp.ones((32, 1), jnp.float32)
    degcol = lax.dot_general(degp_ref[...], ones, (((0,), (0,)), ((), ())),
                             preferred_element_type=jnp.float32)
    dinv = lax.rsqrt(degcol + 1.0)
    dinvb_ref[...] = jnp.broadcast_to(dinv, (512, HID))
    hwp_ref[...] = hw_ref[...] * dinv


def _tc_prep(deg_parts, hw):
    return pl.pallas_call(
        _prep_body,
        grid=(NPAD // 512,),
        in_specs=[
            pl.BlockSpec((32, 512), lambda i: (0, i)),
            pl.BlockSpec((512, HID), lambda i: (i, 0)),
        ],
        out_specs=[
            pl.BlockSpec((512, HID), lambda i: (i, 0)),
            pl.BlockSpec((512, HID), lambda i: (i, 0)),
        ],
        out_shape=[
            jax.ShapeDtypeStruct((NPAD, HID), jnp.float32),
            jax.ShapeDtypeStruct((NPAD, HID), jnp.float32),
        ],
    )(deg_parts, hw)


def _mid_body(sacc_ref, hwp_ref, dinvb_ref, w2_ref, b1_ref, h2wp_ref):
    h1 = dinvb_ref[...] * (sacc_ref[...] + hwp_ref[...]) + b1_ref[...]
    hr = jnp.maximum(h1, 0.0)
    h2w = jnp.dot(hr, w2_ref[...], preferred_element_type=jnp.float32)
    h2wp_ref[...] = dinvb_ref[...] * h2w


def _tc_mid(sacc1, hwp, dinvb, W2, b1):
    return pl.pallas_call(
        _mid_body,
        grid=(NPAD // 512,),
        in_specs=[
            pl.BlockSpec((512, HID), lambda i: (i, 0)),
            pl.BlockSpec((512, HID), lambda i: (i, 0)),
            pl.BlockSpec((512, HID), lambda i: (i, 0)),
            pl.BlockSpec((HID, HID), lambda i: (0, 0)),
            pl.BlockSpec((1, HID), lambda i: (0, 0)),
        ],
        out_specs=pl.BlockSpec((512, HID), lambda i: (i, 0)),
        out_shape=jax.ShapeDtypeStruct((NPAD, HID), jnp.float32),
    )(sacc1, hwp, dinvb, W2, b1)


def _pool_body(sacc_ref, h2wp_ref, dinvb_ref, batch_ref, b2_ref,
               sums_ref, cnts_ref):
    i = pl.program_id(0)
    h2 = dinvb_ref[...] * (sacc_ref[...] + h2wp_ref[...]) + b2_ref[...]
    iot = lax.broadcasted_iota(jnp.int32, (512, B), 1)
    P = (batch_ref[...] == iot).astype(jnp.float32)
    psum = lax.dot_general(P, h2, (((0,), (0,)), ((), ())),
                           preferred_element_type=jnp.float32)
    ones = jnp.ones((512, 1), jnp.float32)
    pcnt = lax.dot_general(P, ones, (((0,), (0,)), ((), ())),
                           preferred_element_type=jnp.float32)

    @pl.when(i == 0)
    def _():
        sums_ref[...] = jnp.zeros_like(sums_ref)
        cnts_ref[...] = jnp.zeros_like(cnts_ref)

    sums_ref[...] += psum
    cnts_ref[...] += pcnt


def _tc_pool(sacc2, h2wp, dinvb, batchcol, b2):
    return pl.pallas_call(
        _pool_body,
        grid=(NPAD // 512,),
        in_specs=[
            pl.BlockSpec((512, HID), lambda i: (i, 0)),
            pl.BlockSpec((512, HID), lambda i: (i, 0)),
            pl.BlockSpec((512, HID), lambda i: (i, 0)),
            pl.BlockSpec((512, 1), lambda i: (i, 0)),
            pl.BlockSpec((1, HID), lambda i: (0, 0)),
        ],
        out_specs=[
            pl.BlockSpec((B, HID), lambda i: (0, 0)),
            pl.BlockSpec((B, 1), lambda i: (0, 0)),
        ],
        out_shape=[
            jax.ShapeDtypeStruct((B, HID), jnp.float32),
            jax.ShapeDtypeStruct((B, 1), jnp.float32),
        ],
    )(sacc2, h2wp, dinvb, batchcol, b2)


def _head_body(sums_ref, cnts_ref, fcW_ref, fcb_ref, srows_ref,
               fc1W_ref, fc1b_ref, fc2W_ref, fc2b_ref, fc3W_ref, fc3b_ref,
               out_ref):
    pooled = sums_ref[...] / jnp.maximum(cnts_ref[...], 1.0)
    g = jnp.dot(pooled, fcW_ref[...],
                preferred_element_type=jnp.float32) + fcb_ref[...]
    x1 = srows_ref[pl.ds(0, B), :]
    x2 = srows_ref[pl.ds(B, B), :]
    z = (jnp.dot(x1, fc1W_ref[pl.ds(0, EMB), :],
                 preferred_element_type=jnp.float32)
         + jnp.dot(x2, fc1W_ref[pl.ds(EMB, EMB), :],
                   preferred_element_type=jnp.float32)
         + jnp.dot(g, fc1W_ref[pl.ds(2 * EMB, EMB), :],
                   preferred_element_type=jnp.float32)
         + fc1b_ref[...])
    z = jnp.maximum(z, 0.0)
    z = jnp.maximum(jnp.dot(z, fc2W_ref[...],
                            preferred_element_type=jnp.float32)
                    + fc2b_ref[...], 0.0)
    out_ref[...] = jnp.dot(z, fc3W_ref[...],
                           preferred_element_type=jnp.float32) + fc3b_ref[...]


def _tc_head(sums, cnts, fcW, fcb, srows, fc1W, fc1b, fc2W, fc2b, fc3W, fc3b):
    return pl.pallas_call(
        _head_body,
        out_shape=jax.ShapeDtypeStruct((B, HID), jnp.float32),
    )(sums, cnts, fcW, fcb, srows, fc1W, fc1b, fc2W, fc2b, fc3W, fc3b)


@jax.jit
def kernel(state, x, edge_index, edge_weight, batch, emb_table,
           W1, b1, W2, b2, fcW, fcb, fc1W, fc1b, fc2W, fc2b, fc3W, fc3b):
    src = edge_index[0]
    dst = edge_index[1]
    srcp = jnp.pad(src, (0, EPAD - E)).astype(jnp.int32)
    dstp = jnp.pad(dst, (0, EPAD - E)).astype(jnp.int32)
    wp = jnp.pad(edge_weight, (0, EPAD - E))
    xpad = jnp.pad(x, (0, NPAD - N)).astype(jnp.int32)
    batchcol = jnp.pad(batch, (0, NPAD - N),
                       constant_values=B).astype(jnp.int32).reshape(NPAD, 1)
    sidx = jnp.concatenate([state[:, 0], state[:, 1]]).astype(jnp.int32)

    T1 = _t1_matmul(emb_table, W1)
    hw, deg_parts, srows = _sc_pre(T1, xpad, dstp, wp, emb_table, sidx)
    dinvb, hwp = _tc_prep(deg_parts, hw)
    sacc1 = _sc_conv(hwp, srcp, dstp, wp).reshape(NPAD, HID)
    h2wp = _tc_mid(sacc1, hwp, dinvb, W2, b1.reshape(1, HID))
    sacc2 = _sc_conv(h2wp, srcp, dstp, wp).reshape(NPAD, HID)
    sums, cnts = _tc_pool(sacc2, h2wp, dinvb, batchcol, b2.reshape(1, HID))
    out = _tc_head(sums, cnts, fcW, fcb.reshape(1, EMB), srows,
                   fc1W, fc1b.reshape(1, HID), fc2W, fc2b.reshape(1, HID),
                   fc3W, fc3b.reshape(1, HID))
    return out


# speedup vs baseline: 12.1632x; 1.8835x over previous
import functools

import jax
import jax.numpy as jnp
from jax import lax
from jax.experimental import pallas as pl
from jax.experimental.pallas import tpu as pltpu, tpu_sc as plsc

N = 50000
NPAD = 50176
HALF = 25088
ACCR = 25120
E = 800000
EPAD = 802816
VOCAB = 100000
EMB = 256
HID = 64
B = 64

_MESH = plsc.VectorSubcoreMesh(core_axis_name="c", subcore_axis_name="s")
_SC_PARAMS = pltpu.CompilerParams(needs_layout_passes=False,
                                  use_tc_tiling_on_sc=False)


def _t1_body(emb_ref, w1_ref, out_ref):
    out_ref[...] = jnp.dot(emb_ref[...], w1_ref[...],
                           preferred_element_type=jnp.float32)


def _t1_matmul(emb, W1):
    bkv = 400
    return pl.pallas_call(
        _t1_body,
        grid=(VOCAB // bkv,),
        in_specs=[
            pl.BlockSpec((bkv, EMB), lambda i: (i, 0)),
            pl.BlockSpec((EMB, HID), lambda i: (0, 0)),
        ],
        out_specs=pl.BlockSpec((bkv, HID), lambda i: (i, 0)),
        out_shape=jax.ShapeDtypeStruct((VOCAB, HID), jnp.float32),
    )(emb, W1)


def _sc_pre_body(t1_hbm, x_hbm, dst_hbm, w_hbm, emb_hbm, sidx_hbm,
                 hw_hbm, degp_hbm, srows_hbm,
                 deg_v, xidx_v, rows_v, dstb_v, wb_v, sall_v, srow_v, sem):
    wid = lax.axis_index("s") * 2 + lax.axis_index("c")

    def zero_body(i, _):
        deg_v[pl.ds(i * 16, 16)] = jnp.zeros((16,), jnp.float32)
        return 0
    lax.fori_loop(0, NPAD // 16, zero_body, 0)

    ebase = wid * (EPAD // 32)
    nchunk = (EPAD // 32) // 128

    def deg_chunk(g, _):
        base = ebase + g * 128
        pltpu.sync_copy(dst_hbm.at[pl.ds(base, 128)], dstb_v)
        pltpu.sync_copy(w_hbm.at[pl.ds(base, 128)], wb_v)

        def deg_step(j, _):
            idx = dstb_v[pl.ds(j * 16, 16)]
            val = wb_v[pl.ds(j * 16, 16)]
            plsc.addupdate_scatter(deg_v, [idx], val)
            return 0
        lax.fori_loop(0, 8, deg_step, 0)
        return 0
    lax.fori_loop(0, nchunk, deg_chunk, 0)
    pltpu.sync_copy(deg_v, degp_hbm.at[wid])

    rbase = wid * 1568
    for j in range(6):
        xidx_v[pl.ds(1568 + j * 16, 16)] = jnp.zeros((16,), jnp.int32)
    pltpu.sync_copy(x_hbm.at[pl.ds(rbase, 1568)], xidx_v.at[pl.ds(0, 1568)])
    for i in range(13):
        sz = 128 if i < 12 else 32
        pltpu.async_copy(t1_hbm.at[xidx_v.at[pl.ds(i * 128, 128)]],
                         rows_v, sem).wait()
        pltpu.sync_copy(rows_v.at[pl.ds(0, sz)],
                        hw_hbm.at[pl.ds(rbase + i * 128, sz)])

    pltpu.sync_copy(sidx_hbm, sall_v)

    @pl.when(wid < 16)
    def _():
        pltpu.async_copy(emb_hbm.at[sall_v.at[pl.ds(wid * 8, 8)]],
                         srow_v, sem).wait()
        pltpu.sync_copy(srow_v, srows_hbm.at[pl.ds(wid * 8, 8)])


def _sc_pre(T1, xpad, dstp, wp, emb, sidx):
    f = pl.kernel(
        _sc_pre_body,
        out_type=(
            jax.ShapeDtypeStruct((NPAD, HID), jnp.float32),
            jax.ShapeDtypeStruct((32, NPAD), jnp.float32),
            jax.ShapeDtypeStruct((128, EMB), jnp.float32),
        ),
        mesh=_MESH,
        scratch_types=[
            pltpu.VMEM((NPAD,), jnp.float32),
            pltpu.VMEM((1664,), jnp.int32),
            pltpu.VMEM((128, HID), jnp.float32),
            pltpu.VMEM((128,), jnp.int32),
            pltpu.VMEM((128,), jnp.float32),
            pltpu.VMEM((128,), jnp.int32),
            pltpu.VMEM((8, EMB), jnp.float32),
            pltpu.SemaphoreType.DMA,
        ],
        compiler_params=_SC_PARAMS,
    )
    return f(T1, xpad, dstp, wp, emb, sidx)


RSTR = 25120


def _sc_part_body(src_hbm, dst_hbm, w_hbm,
                  psrc_hbm, pw_hbm, pldst_hbm, cnts_hbm,
                  ssrc_v, sw_v, sld_v, srcb_v, dstb_v, wb_v, cbuf_v):
    t = lax.axis_index("s") * 2 + lax.axis_index("c")

    def init(i, _):
        ssrc_v[pl.ds(i * 16, 16)] = jnp.zeros((16,), jnp.int32)
        sw_v[pl.ds(i * 16, 16)] = jnp.zeros((16,), jnp.float32)
        sld_v[pl.ds(i * 16, 16)] = jnp.full((16,), HALF, jnp.int32)
        return 0
    lax.fori_loop(0, 25104 // 16, init, 0)

    ebase = t * (EPAD // 32)

    def super_body(gs, offs):
        base = ebase + gs * 1568
        pltpu.sync_copy(src_hbm.at[pl.ds(base, 1568)], srcb_v)
        pltpu.sync_copy(dst_hbm.at[pl.ds(base, 1568)], dstb_v)
        pltpu.sync_copy(w_hbm.at[pl.ds(base, 1568)], wb_v)

        def step(j, offs):
            off0, off1 = offs
            sv = srcb_v[pl.ds(j * 16, 16)]
            dv = dstb_v[pl.ds(j * 16, 16)]
            wv = wb_v[pl.ds(j * 16, 16)]
            m0 = dv < HALF
            ld = jnp.where(m0, dv, dv - HALF)
            m0i = m0.astype(jnp.int32)
            c0 = plsc.cumsum(m0i)
            rank0 = c0 - m0i
            k0 = jnp.max(c0)
            m1i = 1 - m0i
            rank1 = plsc.cumsum(m1i) - m1i
            off1n = off1 - (16 - k0)
            idx = jnp.where(m0, off0 + rank0, off1n + rank1)
            plsc.store_scatter(ssrc_v, [idx], sv)
            plsc.store_scatter(sw_v, [idx], wv)
            plsc.store_scatter(sld_v, [idx], ld)
            return (off0 + k0, off1n)
        return lax.fori_loop(0, 98, step, offs)

    off0, _ = lax.fori_loop(0, 16, super_body,
                            (jnp.int32(0), jnp.int32(HALF)))
    rb = t * RSTR
    pltpu.sync_copy(ssrc_v.at[pl.ds(0, 25104)], psrc_hbm.at[pl.ds(rb, 25104)])
    pltpu.sync_copy(sw_v.at[pl.ds(0, 25104)], pw_hbm.at[pl.ds(rb, 25104)])
    pltpu.sync_copy(sld_v.at[pl.ds(0, 25104)], pldst_hbm.at[pl.ds(rb, 25104)])
    cbuf_v[...] = jnp.full((16,), off0, jnp.int32)
    pltpu.sync_copy(cbuf_v, cnts_hbm.at[t])


def _sc_part(srcp, dstp, wp):
    f = pl.kernel(
        _sc_part_body,
        out_type=(
            jax.ShapeDtypeStruct((32 * RSTR,), jnp.int32),
            jax.ShapeDtypeStruct((32 * RSTR,), jnp.float32),
            jax.ShapeDtypeStruct((32 * RSTR,), jnp.int32),
            jax.ShapeDtypeStruct((32, 16), jnp.int32),
        ),
        mesh=_MESH,
        scratch_types=[
            pltpu.VMEM((25104,), jnp.int32),
            pltpu.VMEM((25104,), jnp.float32),
            pltpu.VMEM((25104,), jnp.int32),
            pltpu.VMEM((1568,), jnp.int32),
            pltpu.VMEM((1568,), jnp.int32),
            pltpu.VMEM((1568,), jnp.float32),
            pltpu.VMEM((16,), jnp.int32),
        ],
        compiler_params=_SC_PARAMS,
    )
    return f(srcp, dstp, wp)


def _sc_conv_body(hp_hbm, psrc_hbm, pw_hbm, pldst_hbm, cnts_hbm, out_hbm,
                  acc_sh, srcs_v, ws_v, lds_v, rows_v, ldst2_v, cvec_v,
                  z_v, sem):
    c = lax.axis_index("c")
    s = lax.axis_index("s")

    def zb(i, _):
        for q in range(4):
            z_v[i, pl.ds(q * 16, 16)] = jnp.zeros((16,), jnp.float32)
        return 0
    lax.fori_loop(0, 128, zb, 0)
    zrows = ACCR // 16
    for i in range(13):
        sz = 128 if i < 12 else 34
        pltpu.sync_copy(z_v.at[pl.ds(0, sz)],
                        acc_sh.at[pl.ds(s * zrows + i * 128, sz)])
    plsc.subcore_barrier()

    isasc = c == 0
    for rr in range(2):
        region = s * 2 + rr
        pltpu.sync_copy(cnts_hbm.at[region], cvec_v)
        M = jnp.max(cvec_v[...])
        count = jnp.where(isasc, M, HALF - M)
        nch = (count + 127) // 128
        nsup = (nch + 23) // 24
        tbase = region * RSTR

        _conv_region(hp_hbm, psrc_hbm, pw_hbm, pldst_hbm, acc_sh,
                     srcs_v, ws_v, lds_v, rows_v, ldst2_v, sem,
                     isasc, M, nch, nsup, tbase)
    plsc.subcore_barrier()

    pltpu.sync_copy(acc_sh.at[pl.ds(s * 1568, 1568)],
                    out_hbm.at[c, pl.ds(s * 1568, 1568)])


def _conv_region(hp_hbm, psrc_hbm, pw_hbm, pldst_hbm, acc_sh,
                 srcs_v, ws_v, lds_v, rows_v, ldst2_v, sem,
                 isasc, M, nch, nsup, tbase):
    def super_body(gs, _):
        sb_a = jnp.minimum(3072 * gs, 22032)
        sb_d = jnp.maximum(HALF - 3072 * (gs + 1), 0)
        sbase = pl.multiple_of(jnp.where(isasc, sb_a, sb_d), 8)
        pltpu.sync_copy(psrc_hbm.at[pl.ds(tbase + sbase, 3072)], srcs_v)
        pltpu.sync_copy(pw_hbm.at[pl.ds(tbase + sbase, 3072)], ws_v)
        pltpu.sync_copy(pldst_hbm.at[pl.ds(tbase + sbase, 3072)], lds_v)
        n_inner = jnp.minimum(24, nch - 24 * gs)

        def chunk(ci_loc, _):
            ci = 24 * gs + ci_loc
            cstart = jnp.where(isasc, 128 * ci, HALF - 128 * (ci + 1))
            boff = pl.multiple_of(cstart - sbase, 8)
            pltpu.async_copy(hp_hbm.at[srcs_v.at[pl.ds(boff, 128)]],
                             rows_v, sem).wait()

            def scale(r, _):
                wv = plsc.load_gather(
                    ws_v, [jnp.full((16,), boff + r, jnp.int32)])
                for q in range(4):
                    rows_v[r, pl.ds(q * 16, 16)] = (
                        rows_v[r, pl.ds(q * 16, 16)] * wv)
                return 0
            lax.fori_loop(0, 128, scale, 0)

            def lj(j, _):
                pos = cstart + j * 16 + lax.broadcasted_iota(
                    jnp.int32, (16,), 0)
                ld = lds_v[pl.ds(boff + j * 16, 16)]
                ok = (pos < M) == isasc
                ldst2_v[0, pl.ds(j * 16, 16)] = jnp.where(ok, ld, HALF)
                return 0
            lax.fori_loop(0, 8, lj, 0)

            pltpu.sync_copy(rows_v, acc_sh.at[ldst2_v.at[0]], add=True)
            return 0
        lax.fori_loop(0, n_inner, chunk, 0)
        return 0
    lax.fori_loop(0, nsup, super_body, 0)


def _sc_conv(hp, psrc, pw, pldst, cnts):
    f = pl.kernel(
        _sc_conv_body,
        out_type=jax.ShapeDtypeStruct((2, HALF, HID), jnp.float32),
        mesh=_MESH,
        scratch_types=[
            pltpu.VMEM_SHARED((ACCR, HID), jnp.float32),
            pltpu.VMEM((3072,), jnp.int32),
            pltpu.VMEM((3072,), jnp.float32),
            pltpu.VMEM((3072,), jnp.int32),
            pltpu.VMEM((128, HID), jnp.float32),
            pltpu.VMEM((1, 128), jnp.int32),
            pltpu.VMEM((16,), jnp.int32),
            pltpu.VMEM((128, HID), jnp.float32),
            pltpu.SemaphoreType.DMA,
        ],
        compiler_params=_SC_PARAMS,
    )
    return f(hp, psrc, pw, pldst, cnts)


def _prep_body(degp_ref, hw_ref, dinvb_ref, hwp_ref):
    ones = jnp.ones((32, 1), jnp.float32)
    degcol = lax.dot_general(degp_ref[...], ones, (((0,), (0,)), ((), ())),
                             preferred_element_type=jnp.float32)
    dinv = lax.rsqrt(degcol + 1.0)
    dinvb_ref[...] = jnp.broadcast_to(dinv, (512, HID))
    hwp_ref[...] = hw_ref[...] * dinv


def _tc_prep(deg_parts, hw):
    return pl.pallas_call(
        _prep_body,
        grid=(NPAD // 512,),
        in_specs=[
            pl.BlockSpec((32, 512), lambda i: (0, i)),
            pl.BlockSpec((512, HID), lambda i: (i, 0)),
        ],
        out_specs=[
            pl.BlockSpec((512, HID), lambda i: (i, 0)),
            pl.BlockSpec((512, HID), lambda i: (i, 0)),
        ],
        out_shape=[
            jax.ShapeDtypeStruct((NPAD, HID), jnp.float32),
            jax.ShapeDtypeStruct((NPAD, HID), jnp.float32),
        ],
    )(deg_parts, hw)


def _mid_body(sacc_ref, hwp_ref, dinvb_ref, w2_ref, b1_ref, h2wp_ref):
    h1 = dinvb_ref[...] * (sacc_ref[...] + hwp_ref[...]) + b1_ref[...]
    hr = jnp.maximum(h1, 0.0)
    h2w = jnp.dot(hr, w2_ref[...], preferred_element_type=jnp.float32)
    h2wp_ref[...] = dinvb_ref[...] * h2w


def _tc_mid(sacc1, hwp, dinvb, W2, b1):
    return pl.pallas_call(
        _mid_body,
        grid=(NPAD // 512,),
        in_specs=[
            pl.BlockSpec((512, HID), lambda i: (i, 0)),
            pl.BlockSpec((512, HID), lambda i: (i, 0)),
            pl.BlockSpec((512, HID), lambda i: (i, 0)),
            pl.BlockSpec((HID, HID), lambda i: (0, 0)),
            pl.BlockSpec((1, HID), lambda i: (0, 0)),
        ],
        out_specs=pl.BlockSpec((512, HID), lambda i: (i, 0)),
        out_shape=jax.ShapeDtypeStruct((NPAD, HID), jnp.float32),
    )(sacc1, hwp, dinvb, W2, b1)


def _pool_body(sacc_ref, h2wp_ref, dinvb_ref, batch_ref, b2_ref,
               sums_ref, cnts_ref):
    i = pl.program_id(0)
    h2 = dinvb_ref[...] * (sacc_ref[...] + h2wp_ref[...]) + b2_ref[...]
    iot = lax.broadcasted_iota(jnp.int32, (512, B), 1)
    P = (batch_ref[...] == iot).astype(jnp.float32)
    psum = lax.dot_general(P, h2, (((0,), (0,)), ((), ())),
                           preferred_element_type=jnp.float32)
    ones = jnp.ones((512, 1), jnp.float32)
    pcnt = lax.dot_general(P, ones, (((0,), (0,)), ((), ())),
                           preferred_element_type=jnp.float32)

    @pl.when(i == 0)
    def _():
        sums_ref[...] = jnp.zeros_like(sums_ref)
        cnts_ref[...] = jnp.zeros_like(cnts_ref)

    sums_ref[...] += psum
    cnts_ref[...] += pcnt


def _tc_pool(sacc2, h2wp, dinvb, batchcol, b2):
    return pl.pallas_call(
        _pool_body,
        grid=(NPAD // 512,),
        in_specs=[
            pl.BlockSpec((512, HID), lambda i: (i, 0)),
            pl.BlockSpec((512, HID), lambda i: (i, 0)),
            pl.BlockSpec((512, HID), lambda i: (i, 0)),
            pl.BlockSpec((512, 1), lambda i: (i, 0)),
            pl.BlockSpec((1, HID), lambda i: (0, 0)),
        ],
        out_specs=[
            pl.BlockSpec((B, HID), lambda i: (0, 0)),
            pl.BlockSpec((B, 1), lambda i: (0, 0)),
        ],
        out_shape=[
            jax.ShapeDtypeStruct((B, HID), jnp.float32),
            jax.ShapeDtypeStruct((B, 1), jnp.float32),
        ],
    )(sacc2, h2wp, dinvb, batchcol, b2)


def _head_body(sums_ref, cnts_ref, fcW_ref, fcb_ref, srows_ref,
               fc1W_ref, fc1b_ref, fc2W_ref, fc2b_ref, fc3W_ref, fc3b_ref,
               out_ref):
    pooled = sums_ref[...] / jnp.maximum(cnts_ref[...], 1.0)
    g = jnp.dot(pooled, fcW_ref[...],
                preferred_element_type=jnp.float32) + fcb_ref[...]
    x1 = srows_ref[pl.ds(0, B), :]
    x2 = srows_ref[pl.ds(B, B), :]
    z = (jnp.dot(x1, fc1W_ref[pl.ds(0, EMB), :],
                 preferred_element_type=jnp.float32)
         + jnp.dot(x2, fc1W_ref[pl.ds(EMB, EMB), :],
                   preferred_element_type=jnp.float32)
         + jnp.dot(g, fc1W_ref[pl.ds(2 * EMB, EMB), :],
                   preferred_element_type=jnp.float32)
         + fc1b_ref[...])
    z = jnp.maximum(z, 0.0)
    z = jnp.maximum(jnp.dot(z, fc2W_ref[...],
                            preferred_element_type=jnp.float32)
                    + fc2b_ref[...], 0.0)
    out_ref[...] = jnp.dot(z, fc3W_ref[...],
                           preferred_element_type=jnp.float32) + fc3b_ref[...]


def _tc_head(sums, cnts, fcW, fcb, srows, fc1W, fc1b, fc2W, fc2b, fc3W, fc3b):
    return pl.pallas_call(
        _head_body,
        out_shape=jax.ShapeDtypeStruct((B, HID), jnp.float32),
    )(sums, cnts, fcW, fcb, srows, fc1W, fc1b, fc2W, fc2b, fc3W, fc3b)


@jax.jit
def kernel(state, x, edge_index, edge_weight, batch, emb_table,
           W1, b1, W2, b2, fcW, fcb, fc1W, fc1b, fc2W, fc2b, fc3W, fc3b):
    src = edge_index[0]
    dst = edge_index[1]
    srcp = jnp.pad(src, (0, EPAD - E)).astype(jnp.int32)
    dstp = jnp.pad(dst, (0, EPAD - E)).astype(jnp.int32)
    wp = jnp.pad(edge_weight, (0, EPAD - E))
    xpad = jnp.pad(x, (0, NPAD - N)).astype(jnp.int32)
    batchcol = jnp.pad(batch, (0, NPAD - N),
                       constant_values=B).astype(jnp.int32).reshape(NPAD, 1)
    sidx = jnp.concatenate([state[:, 0], state[:, 1]]).astype(jnp.int32)

    T1 = _t1_matmul(emb_table, W1)
    hw, deg_parts, srows = _sc_pre(T1, xpad, dstp, wp, emb_table, sidx)
    psrc, pw, pldst, cnts = _sc_part(srcp, dstp, wp)
    dinvb, hwp = _tc_prep(deg_parts, hw)
    sacc1 = _sc_conv(hwp, psrc, pw, pldst, cnts).reshape(NPAD, HID)
    h2wp = _tc_mid(sacc1, hwp, dinvb, W2, b1.reshape(1, HID))
    sacc2 = _sc_conv(h2wp, psrc, pw, pldst, cnts).reshape(NPAD, HID)
    sums, cnts = _tc_pool(sacc2, h2wp, dinvb, batchcol, b2.reshape(1, HID))
    out = _tc_head(sums, cnts, fcW, fcb.reshape(1, EMB), srows,
                   fc1W, fc1b.reshape(1, HID), fc2W, fc2b.reshape(1, HID),
                   fc3W, fc3b.reshape(1, HID))
    return out


# speedup vs baseline: 14.2682x; 1.1731x over previous
import functools

import jax
import jax.numpy as jnp
from jax import lax
from jax.experimental import pallas as pl
from jax.experimental.pallas import tpu as pltpu, tpu_sc as plsc

N = 50000
NPAD = 50176
HALF = 25088
ACCR = 25120
E = 800000
EPAD = 802816
VOCAB = 100000
EMB = 256
HID = 64
B = 64

_MESH = plsc.VectorSubcoreMesh(core_axis_name="c", subcore_axis_name="s")
_SC_PARAMS = pltpu.CompilerParams(needs_layout_passes=False,
                                  use_tc_tiling_on_sc=False)


def _t1_body(emb_ref, w1_ref, out_ref):
    out_ref[...] = jnp.dot(emb_ref[...], w1_ref[...],
                           preferred_element_type=jnp.float32)


def _t1_matmul(emb, W1):
    bkv = 400
    return pl.pallas_call(
        _t1_body,
        grid=(VOCAB // bkv,),
        in_specs=[
            pl.BlockSpec((bkv, EMB), lambda i: (i, 0)),
            pl.BlockSpec((EMB, HID), lambda i: (0, 0)),
        ],
        out_specs=pl.BlockSpec((bkv, HID), lambda i: (i, 0)),
        out_shape=jax.ShapeDtypeStruct((VOCAB, HID), jnp.float32),
    )(emb, W1)


def _sc_pre_body(t1_hbm, x_hbm, dst_hbm, w_hbm, emb_hbm, sidx_hbm,
                 hw_hbm, degp_hbm, srows_hbm,
                 deg_v, xidx_v, rows_v, dstb_v, wb_v, sall_v, srow_v, sem):
    wid = lax.axis_index("s") * 2 + lax.axis_index("c")

    def zero_body(i, _):
        deg_v[pl.ds(i * 16, 16)] = jnp.zeros((16,), jnp.float32)
        return 0
    lax.fori_loop(0, NPAD // 16, zero_body, 0)

    ebase = wid * (EPAD // 32)

    def deg_chunk(g, _):
        base = ebase + g * 1568
        pltpu.sync_copy(dst_hbm.at[pl.ds(base, 1568)], dstb_v)
        pltpu.sync_copy(w_hbm.at[pl.ds(base, 1568)], wb_v)

        def deg_step(j, _):
            idx = dstb_v[pl.ds(j * 16, 16)]
            val = wb_v[pl.ds(j * 16, 16)]
            plsc.addupdate_scatter(deg_v, [idx], val)
            return 0
        lax.fori_loop(0, 98, deg_step, 0)
        return 0
    lax.fori_loop(0, 16, deg_chunk, 0)
    pltpu.sync_copy(deg_v, degp_hbm.at[wid])

    rbase = wid * 1568
    for j in range(6):
        xidx_v[pl.ds(1568 + j * 16, 16)] = jnp.zeros((16,), jnp.int32)
    pltpu.sync_copy(x_hbm.at[pl.ds(rbase, 1568)], xidx_v.at[pl.ds(0, 1568)])
    for i in range(13):
        sz = 128 if i < 12 else 32
        pltpu.async_copy(t1_hbm.at[xidx_v.at[pl.ds(i * 128, 128)]],
                         rows_v, sem).wait()
        pltpu.sync_copy(rows_v.at[pl.ds(0, sz)],
                        hw_hbm.at[pl.ds(rbase + i * 128, sz)])

    pltpu.sync_copy(sidx_hbm, sall_v)

    @pl.when(wid < 16)
    def _():
        pltpu.async_copy(emb_hbm.at[sall_v.at[pl.ds(wid * 8, 8)]],
                         srow_v, sem).wait()
        pltpu.sync_copy(srow_v, srows_hbm.at[pl.ds(wid * 8, 8)])


def _sc_pre(T1, xpad, dstp, wp, emb, sidx):
    f = pl.kernel(
        _sc_pre_body,
        out_type=(
            jax.ShapeDtypeStruct((NPAD, HID), jnp.float32),
            jax.ShapeDtypeStruct((32, NPAD), jnp.float32),
            jax.ShapeDtypeStruct((128, EMB), jnp.float32),
        ),
        mesh=_MESH,
        scratch_types=[
            pltpu.VMEM((NPAD,), jnp.float32),
            pltpu.VMEM((1664,), jnp.int32),
            pltpu.VMEM((128, HID), jnp.float32),
            pltpu.VMEM((1568,), jnp.int32),
            pltpu.VMEM((1568,), jnp.float32),
            pltpu.VMEM((128,), jnp.int32),
            pltpu.VMEM((8, EMB), jnp.float32),
            pltpu.SemaphoreType.DMA,
        ],
        compiler_params=_SC_PARAMS,
    )
    return f(T1, xpad, dstp, wp, emb, sidx)


RSTR = 25120


def _sc_part_body(src_hbm, dst_hbm, w_hbm,
                  psrc_hbm, pw_hbm, pldst_hbm, cnts_hbm,
                  ssrc_v, sw_v, sld_v, srcb_v, dstb_v, wb_v, cbuf_v):
    t = lax.axis_index("s") * 2 + lax.axis_index("c")

    def init(i, _):
        ssrc_v[pl.ds(i * 16, 16)] = jnp.zeros((16,), jnp.int32)
        sw_v[pl.ds(i * 16, 16)] = jnp.zeros((16,), jnp.float32)
        sld_v[pl.ds(i * 16, 16)] = jnp.full((16,), HALF, jnp.int32)
        return 0
    lax.fori_loop(0, 25104 // 16, init, 0)

    ebase = t * (EPAD // 32)

    def super_body(gs, offs):
        base = ebase + gs * 1568
        pltpu.sync_copy(src_hbm.at[pl.ds(base, 1568)], srcb_v)
        pltpu.sync_copy(dst_hbm.at[pl.ds(base, 1568)], dstb_v)
        pltpu.sync_copy(w_hbm.at[pl.ds(base, 1568)], wb_v)

        def step(j, offs):
            off0, off1 = offs
            sv = srcb_v[pl.ds(j * 16, 16)]
            dv = dstb_v[pl.ds(j * 16, 16)]
            wv = wb_v[pl.ds(j * 16, 16)]
            m0 = dv < HALF
            ld = jnp.where(m0, dv, dv - HALF)
            m0i = m0.astype(jnp.int32)
            c0 = plsc.cumsum(m0i)
            rank0 = c0 - m0i
            k0 = jnp.max(c0)
            m1i = 1 - m0i
            rank1 = plsc.cumsum(m1i) - m1i
            off1n = off1 - (16 - k0)
            idx = jnp.where(m0, off0 + rank0, off1n + rank1)
            plsc.store_scatter(ssrc_v, [idx], sv)
            plsc.store_scatter(sw_v, [idx], wv)
            plsc.store_scatter(sld_v, [idx], ld)
            return (off0 + k0, off1n)
        return lax.fori_loop(0, 98, step, offs)

    off0, _ = lax.fori_loop(0, 16, super_body,
                            (jnp.int32(0), jnp.int32(HALF)))
    rb = t * RSTR
    pltpu.sync_copy(ssrc_v.at[pl.ds(0, 25104)], psrc_hbm.at[pl.ds(rb, 25104)])
    pltpu.sync_copy(sw_v.at[pl.ds(0, 25104)], pw_hbm.at[pl.ds(rb, 25104)])
    pltpu.sync_copy(sld_v.at[pl.ds(0, 25104)], pldst_hbm.at[pl.ds(rb, 25104)])
    cbuf_v[...] = jnp.full((16,), off0, jnp.int32)
    pltpu.sync_copy(cbuf_v, cnts_hbm.at[t])


def _sc_part(srcp, dstp, wp):
    f = pl.kernel(
        _sc_part_body,
        out_type=(
            jax.ShapeDtypeStruct((32 * RSTR,), jnp.int32),
            jax.ShapeDtypeStruct((32 * RSTR,), jnp.float32),
            jax.ShapeDtypeStruct((32 * RSTR,), jnp.int32),
            jax.ShapeDtypeStruct((32, 16), jnp.int32),
        ),
        mesh=_MESH,
        scratch_types=[
            pltpu.VMEM((25104,), jnp.int32),
            pltpu.VMEM((25104,), jnp.float32),
            pltpu.VMEM((25104,), jnp.int32),
            pltpu.VMEM((1568,), jnp.int32),
            pltpu.VMEM((1568,), jnp.int32),
            pltpu.VMEM((1568,), jnp.float32),
            pltpu.VMEM((16,), jnp.int32),
        ],
        compiler_params=_SC_PARAMS,
    )
    return f(srcp, dstp, wp)


def _sc_conv_body(hp_hbm, psrc_hbm, pw_hbm, pldst_hbm, cnts_hbm, out_hbm,
                  acc_sh, srcs_v, ws_v, lds_v, rowsA_v, rowsB_v, ldst2_v,
                  cvec_v, gsA, gsB, ssA, ssB):
    c = lax.axis_index("c")
    s = lax.axis_index("s")

    def zb(i, _):
        for q in range(4):
            rowsA_v[i, pl.ds(q * 16, 16)] = jnp.zeros((16,), jnp.float32)
        return 0
    lax.fori_loop(0, 128, zb, 0)
    zrows = ACCR // 16
    for i in range(13):
        sz = 128 if i < 12 else 34
        pltpu.sync_copy(rowsA_v.at[pl.ds(0, sz)],
                        acc_sh.at[pl.ds(s * zrows + i * 128, sz)])
    plsc.subcore_barrier()

    isasc = c == 0
    for rr in range(2):
        region = s * 2 + rr
        pltpu.sync_copy(cnts_hbm.at[region], cvec_v)
        M = jnp.max(cvec_v[...])
        count = jnp.where(isasc, M, HALF - M)
        nch = (count + 127) // 128
        nsup = (nch + 23) // 24
        tbase = region * RSTR

        _conv_region(hp_hbm, psrc_hbm, pw_hbm, pldst_hbm, acc_sh,
                     srcs_v, ws_v, lds_v, rowsA_v, rowsB_v, ldst2_v,
                     gsA, gsB, ssA, ssB, isasc, M, nch, nsup, tbase)
    plsc.subcore_barrier()

    pltpu.sync_copy(acc_sh.at[pl.ds(s * 1568, 1568)],
                    out_hbm.at[c, pl.ds(s * 1568, 1568)])


def _conv_region(hp_hbm, psrc_hbm, pw_hbm, pldst_hbm, acc_sh,
                 srcs_v, ws_v, lds_v, rowsA_v, rowsB_v, ldst2_v,
                 gsA, gsB, ssA, ssB, isasc, M, nch, nsup, tbase):
    iota16 = lax.broadcasted_iota(jnp.int32, (16,), 0)

    def super_body(gs_i, _):
        sb_a = jnp.minimum(3072 * gs_i, 22032)
        sb_d = jnp.maximum(HALF - 3072 * (gs_i + 1), 0)
        sbase = pl.multiple_of(jnp.where(isasc, sb_a, sb_d), 8)
        pltpu.sync_copy(psrc_hbm.at[pl.ds(tbase + sbase, 3072)], srcs_v)
        pltpu.sync_copy(pw_hbm.at[pl.ds(tbase + sbase, 3072)], ws_v)
        pltpu.sync_copy(pldst_hbm.at[pl.ds(tbase + sbase, 3072)], lds_v)
        n_inner = jnp.minimum(24, nch - 24 * gs_i)

        def boff_of(lc):
            ci = 24 * gs_i + lc
            cstart = jnp.where(isasc, 128 * ci, HALF - 128 * (ci + 1))
            return cstart, pl.multiple_of(cstart - sbase, 8)

        def prep(lc, slot):
            cstart, boff = boff_of(lc)

            def lj(j, _):
                pos = cstart + j * 16 + iota16
                ld = lds_v[pl.ds(boff + j * 16, 16)]
                ok = (pos < M) == isasc
                ldst2_v[slot, pl.ds(j * 16, 16)] = jnp.where(ok, ld, HALF)
                return 0
            lax.fori_loop(0, 8, lj, 0)
            return boff

        def gather(boff, rows_x, sem_x):
            pltpu.async_copy(hp_hbm.at[srcs_v.at[pl.ds(boff, 128)]],
                             rows_x, sem_x)

        def wait_dma(rows_x, sem_x):
            pltpu.make_async_copy(hp_hbm.at[pl.ds(0, 128)],
                                  rows_x, sem_x).wait()

        def scale(lc, rows_x):
            _, boff = boff_of(lc)

            def sc(r, _):
                wv = plsc.load_gather(
                    ws_v, [jnp.full((16,), boff + r, jnp.int32)])
                for q in range(4):
                    rows_x[r, pl.ds(q * 16, 16)] = (
                        rows_x[r, pl.ds(q * 16, 16)] * wv)
                return 0
            lax.fori_loop(0, 128, sc, 0)

        def scatter(rows_x, slot, sem_x):
            pltpu.async_copy(rows_x, acc_sh.at[ldst2_v.at[slot]],
                             sem_x, add=True)

        b0 = prep(0, 0)
        gather(b0, rowsA_v, gsA)

        @pl.when(n_inner >= 2)
        def _():
            b1 = prep(1, 1)
            gather(b1, rowsB_v, gsB)

        def pair(p, _):
            la = 2 * p
            lb = la + 1

            @pl.when(la < n_inner)
            def _():
                wait_dma(rowsA_v, gsA)
                scale(la, rowsA_v)

                @pl.when((la > 0) & (lb < n_inner))
                def _():
                    wait_dma(rowsB_v, ssB)
                    bb = prep(lb, 1)
                    gather(bb, rowsB_v, gsB)
                scatter(rowsA_v, 0, ssA)

            @pl.when(lb < n_inner)
            def _():
                wait_dma(rowsB_v, gsB)
                scale(lb, rowsB_v)

                @pl.when(lb + 1 < n_inner)
                def _():
                    wait_dma(rowsA_v, ssA)
                    ba = prep(lb + 1, 0)
                    gather(ba, rowsA_v, gsA)
                scatter(rowsB_v, 1, ssB)
            return 0
        lax.fori_loop(0, 12, pair, 0)

        wait_dma(rowsA_v, ssA)

        @pl.when(n_inner >= 2)
        def _():
            wait_dma(rowsB_v, ssB)
        return 0
    lax.fori_loop(0, nsup, super_body, 0)


def _sc_conv(hp, psrc, pw, pldst, cnts):
    f = pl.kernel(
        _sc_conv_body,
        out_type=jax.ShapeDtypeStruct((2, HALF, HID), jnp.float32),
        mesh=_MESH,
        scratch_types=[
            pltpu.VMEM_SHARED((ACCR, HID), jnp.float32),
            pltpu.VMEM((3072,), jnp.int32),
            pltpu.VMEM((3072,), jnp.float32),
            pltpu.VMEM((3072,), jnp.int32),
            pltpu.VMEM((128, HID), jnp.float32),
            pltpu.VMEM((128, HID), jnp.float32),
            pltpu.VMEM((2, 128), jnp.int32),
            pltpu.VMEM((16,), jnp.int32),
            pltpu.SemaphoreType.DMA,
            pltpu.SemaphoreType.DMA,
            pltpu.SemaphoreType.DMA,
            pltpu.SemaphoreType.DMA,
        ],
        compiler_params=_SC_PARAMS,
    )
    return f(hp, psrc, pw, pldst, cnts)


def _prep_body(degp_ref, hw_ref, dinvb_ref, hwp_ref):
    ones = jnp.ones((32, 1), jnp.float32)
    degcol = lax.dot_general(degp_ref[...], ones, (((0,), (0,)), ((), ())),
                             preferred_element_type=jnp.float32)
    dinv = lax.rsqrt(degcol + 1.0)
    dinvb_ref[...] = jnp.broadcast_to(dinv, (512, HID))
    hwp_ref[...] = hw_ref[...] * dinv


def _tc_prep(deg_parts, hw):
    return pl.pallas_call(
        _prep_body,
        grid=(NPAD // 512,),
        in_specs=[
            pl.BlockSpec((32, 512), lambda i: (0, i)),
            pl.BlockSpec((512, HID), lambda i: (i, 0)),
        ],
        out_specs=[
            pl.BlockSpec((512, HID), lambda i: (i, 0)),
            pl.BlockSpec((512, HID), lambda i: (i, 0)),
        ],
        out_shape=[
            jax.ShapeDtypeStruct((NPAD, HID), jnp.float32),
            jax.ShapeDtypeStruct((NPAD, HID), jnp.float32),
        ],
    )(deg_parts, hw)


def _mid_body(sacc_ref, hwp_ref, dinvb_ref, w2_ref, b1_ref, h2wp_ref):
    h1 = dinvb_ref[...] * (sacc_ref[...] + hwp_ref[...]) + b1_ref[...]
    hr = jnp.maximum(h1, 0.0)
    h2w = jnp.dot(hr, w2_ref[...], preferred_element_type=jnp.float32)
    h2wp_ref[...] = dinvb_ref[...] * h2w


def _tc_mid(sacc1, hwp, dinvb, W2, b1):
    return pl.pallas_call(
        _mid_body,
        grid=(NPAD // 512,),
        in_specs=[
            pl.BlockSpec((512, HID), lambda i: (i, 0)),
            pl.BlockSpec((512, HID), lambda i: (i, 0)),
            pl.BlockSpec((512, HID), lambda i: (i, 0)),
            pl.BlockSpec((HID, HID), lambda i: (0, 0)),
            pl.BlockSpec((1, HID), lambda i: (0, 0)),
        ],
        out_specs=pl.BlockSpec((512, HID), lambda i: (i, 0)),
        out_shape=jax.ShapeDtypeStruct((NPAD, HID), jnp.float32),
    )(sacc1, hwp, dinvb, W2, b1)


def _pool_body(sacc_ref, h2wp_ref, dinvb_ref, batch_ref, b2_ref,
               sums_ref, cnts_ref):
    i = pl.program_id(0)
    h2 = dinvb_ref[...] * (sacc_ref[...] + h2wp_ref[...]) + b2_ref[...]
    iot = lax.broadcasted_iota(jnp.int32, (512, B), 1)
    P = (batch_ref[...] == iot).astype(jnp.float32)
    psum = lax.dot_general(P, h2, (((0,), (0,)), ((), ())),
                           preferred_element_type=jnp.float32)
    ones = jnp.ones((512, 1), jnp.float32)
    pcnt = lax.dot_general(P, ones, (((0,), (0,)), ((), ())),
                           preferred_element_type=jnp.float32)

    @pl.when(i == 0)
    def _():
        sums_ref[...] = jnp.zeros_like(sums_ref)
        cnts_ref[...] = jnp.zeros_like(cnts_ref)

    sums_ref[...] += psum
    cnts_ref[...] += pcnt


def _tc_pool(sacc2, h2wp, dinvb, batchcol, b2):
    return pl.pallas_call(
        _pool_body,
        grid=(NPAD // 512,),
        in_specs=[
            pl.BlockSpec((512, HID), lambda i: (i, 0)),
            pl.BlockSpec((512, HID), lambda i: (i, 0)),
            pl.BlockSpec((512, HID), lambda i: (i, 0)),
            pl.BlockSpec((512, 1), lambda i: (i, 0)),
            pl.BlockSpec((1, HID), lambda i: (0, 0)),
        ],
        out_specs=[
            pl.BlockSpec((B, HID), lambda i: (0, 0)),
            pl.BlockSpec((B, 1), lambda i: (0, 0)),
        ],
        out_shape=[
            jax.ShapeDtypeStruct((B, HID), jnp.float32),
            jax.ShapeDtypeStruct((B, 1), jnp.float32),
        ],
    )(sacc2, h2wp, dinvb, batchcol, b2)


def _head_body(sums_ref, cnts_ref, fcW_ref, fcb_ref, srows_ref,
               fc1W_ref, fc1b_ref, fc2W_ref, fc2b_ref, fc3W_ref, fc3b_ref,
               out_ref):
    pooled = sums_ref[...] / jnp.maximum(cnts_ref[...], 1.0)
    g = jnp.dot(pooled, fcW_ref[...],
                preferred_element_type=jnp.float32) + fcb_ref[...]
    x1 = srows_ref[pl.ds(0, B), :]
    x2 = srows_ref[pl.ds(B, B), :]
    z = (jnp.dot(x1, fc1W_ref[pl.ds(0, EMB), :],
                 preferred_element_type=jnp.float32)
         + jnp.dot(x2, fc1W_ref[pl.ds(EMB, EMB), :],
                   preferred_element_type=jnp.float32)
         + jnp.dot(g, fc1W_ref[pl.ds(2 * EMB, EMB), :],
                   preferred_element_type=jnp.float32)
         + fc1b_ref[...])
    z = jnp.maximum(z, 0.0)
    z = jnp.maximum(jnp.dot(z, fc2W_ref[...],
                            preferred_element_type=jnp.float32)
                    + fc2b_ref[...], 0.0)
    out_ref[...] = jnp.dot(z, fc3W_ref[...],
                           preferred_element_type=jnp.float32) + fc3b_ref[...]


def _tc_head(sums, cnts, fcW, fcb, srows, fc1W, fc1b, fc2W, fc2b, fc3W, fc3b):
    return pl.pallas_call(
        _head_body,
        out_shape=jax.ShapeDtypeStruct((B, HID), jnp.float32),
    )(sums, cnts, fcW, fcb, srows, fc1W, fc1b, fc2W, fc2b, fc3W, fc3b)


@jax.jit
def kernel(state, x, edge_index, edge_weight, batch, emb_table,
           W1, b1, W2, b2, fcW, fcb, fc1W, fc1b, fc2W, fc2b, fc3W, fc3b):
    src = edge_index[0]
    dst = edge_index[1]
    srcp = jnp.pad(src, (0, EPAD - E)).astype(jnp.int32)
    dstp = jnp.pad(dst, (0, EPAD - E)).astype(jnp.int32)
    wp = jnp.pad(edge_weight, (0, EPAD - E))
    xpad = jnp.pad(x, (0, NPAD - N)).astype(jnp.int32)
    batchcol = jnp.pad(batch, (0, NPAD - N),
                       constant_values=B).astype(jnp.int32).reshape(NPAD, 1)
    sidx = jnp.concatenate([state[:, 0], state[:, 1]]).astype(jnp.int32)

    T1 = _t1_matmul(emb_table, W1)
    hw, deg_parts, srows = _sc_pre(T1, xpad, dstp, wp, emb_table, sidx)
    psrc, pw, pldst, cnts = _sc_part(srcp, dstp, wp)
    dinvb, hwp = _tc_prep(deg_parts, hw)
    sacc1 = _sc_conv(hwp, psrc, pw, pldst, cnts).reshape(NPAD, HID)
    h2wp = _tc_mid(sacc1, hwp, dinvb, W2, b1.reshape(1, HID))
    sacc2 = _sc_conv(h2wp, psrc, pw, pldst, cnts).reshape(NPAD, HID)
    sums, cnts = _tc_pool(sacc2, h2wp, dinvb, batchcol, b2.reshape(1, HID))
    out = _tc_head(sums, cnts, fcW, fcb.reshape(1, EMB), srows,
                   fc1W, fc1b.reshape(1, HID), fc2W, fc2b.reshape(1, HID),
                   fc3W, fc3b.reshape(1, HID))
    return out


# speedup vs baseline: 16.9303x; 1.1866x over previous
import functools

import jax
import jax.numpy as jnp
from jax import lax
from jax.experimental import pallas as pl
from jax.experimental.pallas import tpu as pltpu, tpu_sc as plsc

N = 50000
NPAD = 50176
HALF = 25088
ACCR = 25120
E = 800000
EPAD = 802816
VOCAB = 100000
EMB = 256
HID = 64
B = 64

_MESH = plsc.VectorSubcoreMesh(core_axis_name="c", subcore_axis_name="s")
_SC_PARAMS = pltpu.CompilerParams(needs_layout_passes=False,
                                  use_tc_tiling_on_sc=False)


def _t1_body(emb_ref, w1_ref, out_ref):
    out_ref[...] = jnp.dot(emb_ref[...], w1_ref[...],
                           preferred_element_type=jnp.float32)


def _t1_matmul(emb, W1):
    bkv = 400
    return pl.pallas_call(
        _t1_body,
        grid=(VOCAB // bkv,),
        in_specs=[
            pl.BlockSpec((bkv, EMB), lambda i: (i, 0)),
            pl.BlockSpec((EMB, HID), lambda i: (0, 0)),
        ],
        out_specs=pl.BlockSpec((bkv, HID), lambda i: (i, 0)),
        out_shape=jax.ShapeDtypeStruct((VOCAB, HID), jnp.float32),
    )(emb, W1)


def _sc_pre_body(t1_hbm, x_hbm, dst_hbm, w_hbm, emb_hbm, sidx_hbm,
                 hw_hbm, degp_hbm, srows_hbm,
                 deg_v, xidx_v, rows_v, dstb_v, wb_v, sall_v, srow_v, sem):
    wid = lax.axis_index("s") * 2 + lax.axis_index("c")

    def zero_body(i, _):
        deg_v[pl.ds(i * 16, 16)] = jnp.zeros((16,), jnp.float32)
        return 0
    lax.fori_loop(0, NPAD // 16, zero_body, 0)

    ebase = wid * (EPAD // 32)

    def deg_chunk(g, _):
        base = ebase + g * 1568
        pltpu.sync_copy(dst_hbm.at[pl.ds(base, 1568)], dstb_v)
        pltpu.sync_copy(w_hbm.at[pl.ds(base, 1568)], wb_v)

        @plsc.parallel_loop(0, 98, unroll=4)
        def _(j):
            idx = dstb_v[pl.ds(j * 16, 16)]
            val = wb_v[pl.ds(j * 16, 16)]
            plsc.addupdate_scatter(deg_v, [idx], val)
        return 0
    lax.fori_loop(0, 16, deg_chunk, 0)
    pltpu.sync_copy(deg_v, degp_hbm.at[wid])

    rbase = wid * 1568
    for j in range(6):
        xidx_v[pl.ds(1568 + j * 16, 16)] = jnp.zeros((16,), jnp.int32)
    pltpu.sync_copy(x_hbm.at[pl.ds(rbase, 1568)], xidx_v.at[pl.ds(0, 1568)])
    for i in range(13):
        sz = 128 if i < 12 else 32
        pltpu.async_copy(t1_hbm.at[xidx_v.at[pl.ds(i * 128, 128)]],
                         rows_v, sem).wait()
        pltpu.sync_copy(rows_v.at[pl.ds(0, sz)],
                        hw_hbm.at[pl.ds(rbase + i * 128, sz)])

    pltpu.sync_copy(sidx_hbm, sall_v)

    @pl.when(wid < 16)
    def _():
        pltpu.async_copy(emb_hbm.at[sall_v.at[pl.ds(wid * 8, 8)]],
                         srow_v, sem).wait()
        pltpu.sync_copy(srow_v, srows_hbm.at[pl.ds(wid * 8, 8)])


def _sc_pre(T1, xpad, dstp, wp, emb, sidx):
    f = pl.kernel(
        _sc_pre_body,
        out_type=(
            jax.ShapeDtypeStruct((NPAD, HID), jnp.float32),
            jax.ShapeDtypeStruct((32, NPAD), jnp.float32),
            jax.ShapeDtypeStruct((128, EMB), jnp.float32),
        ),
        mesh=_MESH,
        scratch_types=[
            pltpu.VMEM((NPAD,), jnp.float32),
            pltpu.VMEM((1664,), jnp.int32),
            pltpu.VMEM((128, HID), jnp.float32),
            pltpu.VMEM((1568,), jnp.int32),
            pltpu.VMEM((1568,), jnp.float32),
            pltpu.VMEM((128,), jnp.int32),
            pltpu.VMEM((8, EMB), jnp.float32),
            pltpu.SemaphoreType.DMA,
        ],
        compiler_params=_SC_PARAMS,
    )
    return f(T1, xpad, dstp, wp, emb, sidx)


RSTR = 25120


def _sc_part_body(src_hbm, dst_hbm, w_hbm,
                  psrc_hbm, pw_hbm, pldst_hbm, cnts_hbm,
                  ssrc_v, sw_v, sld_v, srcb_v, dstb_v, wb_v, cbuf_v):
    t = lax.axis_index("s") * 2 + lax.axis_index("c")

    @plsc.parallel_loop(0, 25104 // 16, unroll=8)
    def _(i):
        ssrc_v[pl.ds(i * 16, 16)] = jnp.zeros((16,), jnp.int32)
        sw_v[pl.ds(i * 16, 16)] = jnp.zeros((16,), jnp.float32)
        sld_v[pl.ds(i * 16, 16)] = jnp.full((16,), HALF, jnp.int32)

    ebase = t * (EPAD // 32)

    def super_body(gs, offs):
        base = ebase + gs * 1568
        pltpu.sync_copy(src_hbm.at[pl.ds(base, 1568)], srcb_v)
        pltpu.sync_copy(dst_hbm.at[pl.ds(base, 1568)], dstb_v)
        pltpu.sync_copy(w_hbm.at[pl.ds(base, 1568)], wb_v)

        def step(j, offs):
            off0, off1 = offs
            sv = srcb_v[pl.ds(j * 16, 16)]
            dv = dstb_v[pl.ds(j * 16, 16)]
            wv = wb_v[pl.ds(j * 16, 16)]
            m0 = dv < HALF
            ld = jnp.where(m0, dv, dv - HALF)
            m0i = m0.astype(jnp.int32)
            c0 = plsc.cumsum(m0i)
            rank0 = c0 - m0i
            k0 = jnp.max(c0)
            m1i = 1 - m0i
            rank1 = plsc.cumsum(m1i) - m1i
            off1n = off1 - (16 - k0)
            idx = jnp.where(m0, off0 + rank0, off1n + rank1)
            plsc.store_scatter(ssrc_v, [idx], sv)
            plsc.store_scatter(sw_v, [idx], wv)
            plsc.store_scatter(sld_v, [idx], ld)
            return (off0 + k0, off1n)
        return plsc.parallel_loop(0, 98, unroll=2, carry=offs)(step)

    off0, _ = lax.fori_loop(0, 16, super_body,
                            (jnp.int32(0), jnp.int32(HALF)))
    rb = t * RSTR
    pltpu.sync_copy(ssrc_v.at[pl.ds(0, 25104)], psrc_hbm.at[pl.ds(rb, 25104)])
    pltpu.sync_copy(sw_v.at[pl.ds(0, 25104)], pw_hbm.at[pl.ds(rb, 25104)])
    pltpu.sync_copy(sld_v.at[pl.ds(0, 25104)], pldst_hbm.at[pl.ds(rb, 25104)])
    cbuf_v[...] = jnp.full((16,), off0, jnp.int32)
    pltpu.sync_copy(cbuf_v, cnts_hbm.at[t])


def _sc_part(srcp, dstp, wp):
    f = pl.kernel(
        _sc_part_body,
        out_type=(
            jax.ShapeDtypeStruct((32 * RSTR,), jnp.int32),
            jax.ShapeDtypeStruct((32 * RSTR,), jnp.float32),
            jax.ShapeDtypeStruct((32 * RSTR,), jnp.int32),
            jax.ShapeDtypeStruct((32, 16), jnp.int32),
        ),
        mesh=_MESH,
        scratch_types=[
            pltpu.VMEM((25104,), jnp.int32),
            pltpu.VMEM((25104,), jnp.float32),
            pltpu.VMEM((25104,), jnp.int32),
            pltpu.VMEM((1568,), jnp.int32),
            pltpu.VMEM((1568,), jnp.int32),
            pltpu.VMEM((1568,), jnp.float32),
            pltpu.VMEM((16,), jnp.int32),
        ],
        compiler_params=_SC_PARAMS,
    )
    return f(srcp, dstp, wp)


def _sc_conv_body(hp_hbm, psrc_hbm, pw_hbm, pldst_hbm, cnts_hbm, out_hbm,
                  acc_sh, srcs_v, ws_v, lds_v, rowsA_v, rowsB_v, ldst2_v,
                  cvec_v, gsA, gsB, ssA, ssB):
    c = lax.axis_index("c")
    s = lax.axis_index("s")

    @plsc.parallel_loop(0, 128, unroll=8)
    def _(i):
        for q in range(4):
            rowsA_v[i, pl.ds(q * 16, 16)] = jnp.zeros((16,), jnp.float32)
    zrows = ACCR // 16
    for i in range(13):
        sz = 128 if i < 12 else 34
        pltpu.sync_copy(rowsA_v.at[pl.ds(0, sz)],
                        acc_sh.at[pl.ds(s * zrows + i * 128, sz)])
    plsc.subcore_barrier()

    isasc = c == 0
    for rr in range(2):
        region = s * 2 + rr
        pltpu.sync_copy(cnts_hbm.at[region], cvec_v)
        M = jnp.max(cvec_v[...])
        count = jnp.where(isasc, M, HALF - M)
        nch = (count + 127) // 128
        nsup = (nch + 23) // 24
        tbase = region * RSTR

        _conv_region(hp_hbm, psrc_hbm, pw_hbm, pldst_hbm, acc_sh,
                     srcs_v, ws_v, lds_v, rowsA_v, rowsB_v, ldst2_v,
                     gsA, gsB, ssA, ssB, isasc, M, nch, nsup, tbase)
    plsc.subcore_barrier()

    pltpu.sync_copy(acc_sh.at[pl.ds(s * 1568, 1568)],
                    out_hbm.at[c, pl.ds(s * 1568, 1568)])


def _conv_region(hp_hbm, psrc_hbm, pw_hbm, pldst_hbm, acc_sh,
                 srcs_v, ws_v, lds_v, rowsA_v, rowsB_v, ldst2_v,
                 gsA, gsB, ssA, ssB, isasc, M, nch, nsup, tbase):
    iota16 = lax.broadcasted_iota(jnp.int32, (16,), 0)

    def super_body(gs_i, _):
        sb_a = jnp.minimum(3072 * gs_i, 22032)
        sb_d = jnp.maximum(HALF - 3072 * (gs_i + 1), 0)
        sbase = pl.multiple_of(jnp.where(isasc, sb_a, sb_d), 8)
        pltpu.sync_copy(psrc_hbm.at[pl.ds(tbase + sbase, 3072)], srcs_v)
        pltpu.sync_copy(pw_hbm.at[pl.ds(tbase + sbase, 3072)], ws_v)
        pltpu.sync_copy(pldst_hbm.at[pl.ds(tbase + sbase, 3072)], lds_v)
        n_inner = jnp.minimum(24, nch - 24 * gs_i)

        def boff_of(lc):
            ci = 24 * gs_i + lc
            cstart = jnp.where(isasc, 128 * ci, HALF - 128 * (ci + 1))
            return cstart, pl.multiple_of(cstart - sbase, 8)

        def prep(lc, slot):
            cstart, boff = boff_of(lc)

            @plsc.parallel_loop(0, 8, unroll=8)
            def _(j):
                pos = cstart + j * 16 + iota16
                ld = lds_v[pl.ds(boff + j * 16, 16)]
                ok = (pos < M) == isasc
                ldst2_v[slot, pl.ds(j * 16, 16)] = jnp.where(ok, ld, HALF)
            return boff

        def gather(boff, rows_x, sem_x):
            pltpu.async_copy(hp_hbm.at[srcs_v.at[pl.ds(boff, 128)]],
                             rows_x, sem_x)

        def wait_dma(rows_x, sem_x):
            pltpu.make_async_copy(hp_hbm.at[pl.ds(0, 128)],
                                  rows_x, sem_x).wait()

        def scale(lc, rows_x):
            _, boff = boff_of(lc)

            @plsc.parallel_loop(0, 128, unroll=8)
            def _(r):
                wv = plsc.load_gather(
                    ws_v, [jnp.full((16,), boff + r, jnp.int32)])
                for q in range(4):
                    rows_x[r, pl.ds(q * 16, 16)] = (
                        rows_x[r, pl.ds(q * 16, 16)] * wv)

        def scatter(rows_x, slot, sem_x):
            pltpu.async_copy(rows_x, acc_sh.at[ldst2_v.at[slot]],
                             sem_x, add=True)

        b0 = prep(0, 0)
        gather(b0, rowsA_v, gsA)

        @pl.when(n_inner >= 2)
        def _():
            b1 = prep(1, 1)
            gather(b1, rowsB_v, gsB)

        def pair(p, _):
            la = 2 * p
            lb = la + 1

            @pl.when(la < n_inner)
            def _():
                wait_dma(rowsA_v, gsA)
                scale(la, rowsA_v)

                @pl.when((la > 0) & (lb < n_inner))
                def _():
                    wait_dma(rowsB_v, ssB)
                    bb = prep(lb, 1)
                    gather(bb, rowsB_v, gsB)
                scatter(rowsA_v, 0, ssA)

            @pl.when(lb < n_inner)
            def _():
                wait_dma(rowsB_v, gsB)
                scale(lb, rowsB_v)

                @pl.when(lb + 1 < n_inner)
                def _():
                    wait_dma(rowsA_v, ssA)
                    ba = prep(lb + 1, 0)
                    gather(ba, rowsA_v, gsA)
                scatter(rowsB_v, 1, ssB)
            return 0
        lax.fori_loop(0, 12, pair, 0)

        wait_dma(rowsA_v, ssA)

        @pl.when(n_inner >= 2)
        def _():
            wait_dma(rowsB_v, ssB)
        return 0
    lax.fori_loop(0, nsup, super_body, 0)


def _sc_conv(hp, psrc, pw, pldst, cnts):
    f = pl.kernel(
        _sc_conv_body,
        out_type=jax.ShapeDtypeStruct((2, HALF, HID), jnp.float32),
        mesh=_MESH,
        scratch_types=[
            pltpu.VMEM_SHARED((ACCR, HID), jnp.float32),
            pltpu.VMEM((3072,), jnp.int32),
            pltpu.VMEM((3072,), jnp.float32),
            pltpu.VMEM((3072,), jnp.int32),
            pltpu.VMEM((128, HID), jnp.float32),
            pltpu.VMEM((128, HID), jnp.float32),
            pltpu.VMEM((2, 128), jnp.int32),
            pltpu.VMEM((16,), jnp.int32),
            pltpu.SemaphoreType.DMA,
            pltpu.SemaphoreType.DMA,
            pltpu.SemaphoreType.DMA,
            pltpu.SemaphoreType.DMA,
        ],
        compiler_params=_SC_PARAMS,
    )
    return f(hp, psrc, pw, pldst, cnts)


def _prep_body(degp_ref, hw_ref, dinvb_ref, hwp_ref):
    ones = jnp.ones((32, 1), jnp.float32)
    degcol = lax.dot_general(degp_ref[...], ones, (((0,), (0,)), ((), ())),
                             preferred_element_type=jnp.float32)
    dinv = lax.rsqrt(degcol + 1.0)
    dinvb_ref[...] = jnp.broadcast_to(dinv, (512, HID))
    hwp_ref[...] = hw_ref[...] * dinv


def _tc_prep(deg_parts, hw):
    return pl.pallas_call(
        _prep_body,
        grid=(NPAD // 512,),
        in_specs=[
            pl.BlockSpec((32, 512), lambda i: (0, i)),
            pl.BlockSpec((512, HID), lambda i: (i, 0)),
        ],
        out_specs=[
            pl.BlockSpec((512, HID), lambda i: (i, 0)),
            pl.BlockSpec((512, HID), lambda i: (i, 0)),
        ],
        out_shape=[
            jax.ShapeDtypeStruct((NPAD, HID), jnp.float32),
            jax.ShapeDtypeStruct((NPAD, HID), jnp.float32),
        ],
    )(deg_parts, hw)


def _mid_body(sacc_ref, hwp_ref, dinvb_ref, w2_ref, b1_ref, h2wp_ref):
    h1 = dinvb_ref[...] * (sacc_ref[...] + hwp_ref[...]) + b1_ref[...]
    hr = jnp.maximum(h1, 0.0)
    h2w = jnp.dot(hr, w2_ref[...], preferred_element_type=jnp.float32)
    h2wp_ref[...] = dinvb_ref[...] * h2w


def _tc_mid(sacc1, hwp, dinvb, W2, b1):
    return pl.pallas_call(
        _mid_body,
        grid=(NPAD // 512,),
        in_specs=[
            pl.BlockSpec((512, HID), lambda i: (i, 0)),
            pl.BlockSpec((512, HID), lambda i: (i, 0)),
            pl.BlockSpec((512, HID), lambda i: (i, 0)),
            pl.BlockSpec((HID, HID), lambda i: (0, 0)),
            pl.BlockSpec((1, HID), lambda i: (0, 0)),
        ],
        out_specs=pl.BlockSpec((512, HID), lambda i: (i, 0)),
        out_shape=jax.ShapeDtypeStruct((NPAD, HID), jnp.float32),
    )(sacc1, hwp, dinvb, W2, b1)


def _pool_body(sacc_ref, h2wp_ref, dinvb_ref, batch_ref, b2_ref,
               sums_ref, cnts_ref):
    i = pl.program_id(0)
    h2 = dinvb_ref[...] * (sacc_ref[...] + h2wp_ref[...]) + b2_ref[...]
    iot = lax.broadcasted_iota(jnp.int32, (512, B), 1)
    P = (batch_ref[...] == iot).astype(jnp.float32)
    psum = lax.dot_general(P, h2, (((0,), (0,)), ((), ())),
                           preferred_element_type=jnp.float32)
    ones = jnp.ones((512, 1), jnp.float32)
    pcnt = lax.dot_general(P, ones, (((0,), (0,)), ((), ())),
                           preferred_element_type=jnp.float32)

    @pl.when(i == 0)
    def _():
        sums_ref[...] = jnp.zeros_like(sums_ref)
        cnts_ref[...] = jnp.zeros_like(cnts_ref)

    sums_ref[...] += psum
    cnts_ref[...] += pcnt


def _tc_pool(sacc2, h2wp, dinvb, batchcol, b2):
    return pl.pallas_call(
        _pool_body,
        grid=(NPAD // 512,),
        in_specs=[
            pl.BlockSpec((512, HID), lambda i: (i, 0)),
            pl.BlockSpec((512, HID), lambda i: (i, 0)),
            pl.BlockSpec((512, HID), lambda i: (i, 0)),
            pl.BlockSpec((512, 1), lambda i: (i, 0)),
            pl.BlockSpec((1, HID), lambda i: (0, 0)),
        ],
        out_specs=[
            pl.BlockSpec((B, HID), lambda i: (0, 0)),
            pl.BlockSpec((B, 1), lambda i: (0, 0)),
        ],
        out_shape=[
            jax.ShapeDtypeStruct((B, HID), jnp.float32),
            jax.ShapeDtypeStruct((B, 1), jnp.float32),
        ],
    )(sacc2, h2wp, dinvb, batchcol, b2)


def _head_body(sums_ref, cnts_ref, fcW_ref, fcb_ref, srows_ref,
               fc1W_ref, fc1b_ref, fc2W_ref, fc2b_ref, fc3W_ref, fc3b_ref,
               out_ref):
    pooled = sums_ref[...] / jnp.maximum(cnts_ref[...], 1.0)
    g = jnp.dot(pooled, fcW_ref[...],
                preferred_element_type=jnp.float32) + fcb_ref[...]
    x1 = srows_ref[pl.ds(0, B), :]
    x2 = srows_ref[pl.ds(B, B), :]
    z = (jnp.dot(x1, fc1W_ref[pl.ds(0, EMB), :],
                 preferred_element_type=jnp.float32)
         + jnp.dot(x2, fc1W_ref[pl.ds(EMB, EMB), :],
                   preferred_element_type=jnp.float32)
         + jnp.dot(g, fc1W_ref[pl.ds(2 * EMB, EMB), :],
                   preferred_element_type=jnp.float32)
         + fc1b_ref[...])
    z = jnp.maximum(z, 0.0)
    z = jnp.maximum(jnp.dot(z, fc2W_ref[...],
                            preferred_element_type=jnp.float32)
                    + fc2b_ref[...], 0.0)
    out_ref[...] = jnp.dot(z, fc3W_ref[...],
                           preferred_element_type=jnp.float32) + fc3b_ref[...]


def _tc_head(sums, cnts, fcW, fcb, srows, fc1W, fc1b, fc2W, fc2b, fc3W, fc3b):
    return pl.pallas_call(
        _head_body,
        out_shape=jax.ShapeDtypeStruct((B, HID), jnp.float32),
    )(sums, cnts, fcW, fcb, srows, fc1W, fc1b, fc2W, fc2b, fc3W, fc3b)


@jax.jit
def kernel(state, x, edge_index, edge_weight, batch, emb_table,
           W1, b1, W2, b2, fcW, fcb, fc1W, fc1b, fc2W, fc2b, fc3W, fc3b):
    src = edge_index[0]
    dst = edge_index[1]
    srcp = jnp.pad(src, (0, EPAD - E)).astype(jnp.int32)
    dstp = jnp.pad(dst, (0, EPAD - E)).astype(jnp.int32)
    wp = jnp.pad(edge_weight, (0, EPAD - E))
    xpad = jnp.pad(x, (0, NPAD - N)).astype(jnp.int32)
    batchcol = jnp.pad(batch, (0, NPAD - N),
                       constant_values=B).astype(jnp.int32).reshape(NPAD, 1)
    sidx = jnp.concatenate([state[:, 0], state[:, 1]]).astype(jnp.int32)

    T1 = _t1_matmul(emb_table, W1)
    hw, deg_parts, srows = _sc_pre(T1, xpad, dstp, wp, emb_table, sidx)
    psrc, pw, pldst, cnts = _sc_part(srcp, dstp, wp)
    dinvb, hwp = _tc_prep(deg_parts, hw)
    sacc1 = _sc_conv(hwp, psrc, pw, pldst, cnts).reshape(NPAD, HID)
    h2wp = _tc_mid(sacc1, hwp, dinvb, W2, b1.reshape(1, HID))
    sacc2 = _sc_conv(h2wp, psrc, pw, pldst, cnts).reshape(NPAD, HID)
    sums, cnts = _tc_pool(sacc2, h2wp, dinvb, batchcol, b2.reshape(1, HID))
    out = _tc_head(sums, cnts, fcW, fcb.reshape(1, EMB), srows,
                   fc1W, fc1b.reshape(1, HID), fc2W, fc2b.reshape(1, HID),
                   fc3W, fc3b.reshape(1, HID))
    return out
